# Initial kernel scaffold; baseline (speedup 1.0000x reference)
import jax
import jax.numpy as jnp
from jax.experimental import pallas as pl


def kernel(input_ids, embedding, W, b):
    raise NotImplementedError("write your pallas kernel here")


# speedup vs baseline: 11.2474x; 11.2474x over previous
import functools

import jax
import jax.numpy as jnp
from jax import lax
from jax.experimental import pallas as pl
from jax.experimental.pallas import tpu as pltpu
from jax.experimental.pallas import tpu_sc as plsc

_NUM_CLUSTERS = 100000
_DIM = 64
_NUM_LABELS = 128
_PAD_ID = _NUM_CLUSTERS
_B = 4096
_L = 200

_NC = 2
_NS = 16
_NW = _NC * _NS
_ROWS_PER_W = _B // _NW
_LANES = 16


def _sc_gather_sum(ids_flat, embedding):
    mesh = plsc.VectorSubcoreMesh(core_axis_name="c", subcore_axis_name="s")

    @functools.partial(
        pl.kernel,
        mesh=mesh,
        out_type=jax.ShapeDtypeStruct((_B, _DIM), jnp.float32),
        compiler_params=pltpu.CompilerParams(use_tc_tiling_on_sc=False),
        scratch_types=[
            pltpu.VMEM((_ROWS_PER_W * _L,), jnp.int32),
            pltpu.VMEM((_L, _DIM), jnp.float32),
            pltpu.VMEM((_ROWS_PER_W, _DIM), jnp.float32),
            pltpu.SemaphoreType.DMA,
        ],
    )
    def k(ids_hbm, emb_hbm, out_hbm, idx_v, buf_v, acc_v, sem):
        wid = lax.axis_index("s") * _NC + lax.axis_index("c")
        base = wid * _ROWS_PER_W
        pltpu.sync_copy(ids_hbm.at[pl.ds(base * _L, _ROWS_PER_W * _L)], idx_v)

        def row(r, carry):
            cp1 = pltpu.async_copy(
                emb_hbm.at[idx_v.at[pl.ds(r * _L, 128)]],
                buf_v.at[pl.ds(0, 128)], sem)
            cp2 = pltpu.async_copy(
                emb_hbm.at[idx_v.at[pl.ds(r * _L + 128, _L - 128)]],
                buf_v.at[pl.ds(128, _L - 128)], sem)
            cp1.wait()
            cp2.wait()

            def red(j, accs):
                out = []
                for c in range(_DIM // _LANES):
                    a = accs[c]
                    for u in range(4):
                        a = a + buf_v[j * 4 + u, pl.ds(c * _LANES, _LANES)]
                    out.append(a)
                return tuple(out)

            zeros = tuple(
                jnp.zeros((_LANES,), jnp.float32)
                for _ in range(_DIM // _LANES))
            accs = lax.fori_loop(0, _L // 4, red, zeros)
            for c in range(_DIM // _LANES):
                acc_v[r, pl.ds(c * _LANES, _LANES)] = accs[c]
            return carry

        lax.fori_loop(0, _ROWS_PER_W, row, 0)
        pltpu.sync_copy(acc_v, out_hbm.at[pl.ds(base, _ROWS_PER_W)])

    return k(ids_flat, embedding)


def _tc_head(input_ids, emb_sum, W, b2d):
    def body(ids_ref, es_ref, w_ref, b_ref, out_ref):
        ids = ids_ref[...]
        cnt = jnp.sum((ids != _PAD_ID).astype(jnp.float32), axis=1,
                      keepdims=True)
        pooled = es_ref[...] / jnp.maximum(cnt, 1.0)
        out_ref[...] = (
            jnp.dot(pooled, w_ref[...], preferred_element_type=jnp.float32)
            + b_ref[...])

    return pl.pallas_call(
        body,
        out_shape=jax.ShapeDtypeStruct((_B, _NUM_LABELS), jnp.float32),
    )(input_ids, emb_sum, W, b2d)


def kernel(input_ids, embedding, W, b):
    ids = input_ids.astype(jnp.int32)
    emb_sum = _sc_gather_sum(ids.reshape(-1), embedding)
    return _tc_head(ids, emb_sum, W, b.reshape(1, _NUM_LABELS))


# speedup vs baseline: 16.6769x; 1.4827x over previous
import functools

import jax
import jax.numpy as jnp
from jax import lax
from jax.experimental import pallas as pl
from jax.experimental.pallas import tpu as pltpu
from jax.experimental.pallas import tpu_sc as plsc

_NUM_CLUSTERS = 100000
_DIM = 64
_NUM_LABELS = 128
_PAD_ID = _NUM_CLUSTERS
_B = 4096
_L = 200

_NC = 2
_NS = 16
_NW = _NC * _NS
_ROWS_PER_W = _B // _NW
_LANES = 16


def _sc_gather_sum(ids, embedding):
    mesh = plsc.VectorSubcoreMesh(core_axis_name="c", subcore_axis_name="s")

    @functools.partial(
        pl.kernel,
        mesh=mesh,
        out_type=jax.ShapeDtypeStruct((_B, _DIM), jnp.float32),
        compiler_params=pltpu.CompilerParams(use_tc_tiling_on_sc=False),
        scratch_types=[
            pltpu.VMEM((_ROWS_PER_W, _L), jnp.int32),
            pltpu.VMEM((2, _L, _DIM), jnp.float32),
            pltpu.VMEM((_ROWS_PER_W, _DIM), jnp.float32),
            pltpu.SemaphoreType.DMA((2,)),
        ],
    )
    def k(ids_hbm, emb_hbm, out_hbm, idx_v, buf_v, acc_v, sem):
        wid = lax.axis_index("s") * _NC + lax.axis_index("c")
        base = wid * _ROWS_PER_W
        pltpu.sync_copy(ids_hbm.at[pl.ds(base, _ROWS_PER_W)], idx_v)

        def copies(r, par):
            return (
                pltpu.make_async_copy(
                    emb_hbm.at[idx_v.at[r, pl.ds(0, 128)]],
                    buf_v.at[par, pl.ds(0, 128)], sem.at[par]),
                pltpu.make_async_copy(
                    emb_hbm.at[idx_v.at[r, pl.ds(128, _L - 128)]],
                    buf_v.at[par, pl.ds(128, _L - 128)], sem.at[par]),
            )

        def fire(r, par):
            for cp in copies(r, par):
                cp.start()

        def drain(r, par):
            for cp in copies(r, par):
                cp.wait()

        fire(0, 0)

        def row(r, carry):
            par = r & 1

            @pl.when(r < _ROWS_PER_W - 1)
            def _():
                fire(r + 1, 1 - par)

            drain(r, par)

            def red(j, accs):
                out = []
                for c in range(_DIM // _LANES):
                    a = accs[c]
                    for u in range(4):
                        a = a + buf_v[par, j * 4 + u,
                                      pl.ds(c * _LANES, _LANES)]
                    out.append(a)
                return tuple(out)

            zeros = tuple(
                jnp.zeros((_LANES,), jnp.float32)
                for _ in range(_DIM // _LANES))
            accs = lax.fori_loop(0, _L // 4, red, zeros)
            for c in range(_DIM // _LANES):
                acc_v[r, pl.ds(c * _LANES, _LANES)] = accs[c]
            return carry

        lax.fori_loop(0, _ROWS_PER_W, row, 0)
        pltpu.sync_copy(acc_v, out_hbm.at[pl.ds(base, _ROWS_PER_W)])

    return k(ids, embedding)


def _tc_head(input_ids, emb_sum, W, b2d):
    def body(ids_ref, es_ref, w_ref, b_ref, out_ref):
        ids = ids_ref[...]
        cnt = jnp.sum((ids != _PAD_ID).astype(jnp.float32), axis=1,
                      keepdims=True)
        pooled = es_ref[...] / jnp.maximum(cnt, 1.0)
        out_ref[...] = (
            jnp.dot(pooled, w_ref[...], preferred_element_type=jnp.float32)
            + b_ref[...])

    return pl.pallas_call(
        body,
        out_shape=jax.ShapeDtypeStruct((_B, _NUM_LABELS), jnp.float32),
    )(input_ids, emb_sum, W, b2d)


def kernel(input_ids, embedding, W, b):
    ids = input_ids.astype(jnp.int32)
    emb_sum = _sc_gather_sum(ids, embedding)
    return _tc_head(ids, emb_sum, W, b.reshape(1, _NUM_LABELS))
